# Initial kernel scaffold; baseline (speedup 1.0000x reference)
import jax
import jax.numpy as jnp
from jax.experimental import pallas as pl


def kernel(x, embeddings, W, b):
    raise NotImplementedError("write your pallas kernel here")


# speedup vs baseline: 1.0204x; 1.0204x over previous
import functools

import jax
import jax.numpy as jnp
from jax.experimental import pallas as pl
from jax.experimental.pallas import tpu as pltpu

N_POINTS = 200
NUM_LEVELS = 5
BASE_RES = 8
HASHMAP_SIZE = 2 ** 19
PRIMES = (1, 2654435761, 805459861)


def _levels():
    resolutions, sizes, offsets = [], [], [0]
    for l in range(NUM_LEVELS):
        res = BASE_RES * (2 ** l)
        size = min(HASHMAP_SIZE, (res + 1) ** 3)
        resolutions.append(res)
        sizes.append(size)
        offsets.append(offsets[-1] + size)
    return resolutions, sizes, offsets


def _encode_z(x, embeddings, W, b):
    N = x.shape[0]
    p0, p1 = x[:, :2], x[:, 2:4]
    t = jnp.linspace(0.0, 1.0, N_POINTS, dtype=jnp.float32)
    pts = p0[:, None, :] + t[None, :, None] * (p1 - p0)[:, None, :]
    tt = jnp.broadcast_to(t[None, :, None], (N, N_POINTS, 1))
    coords = jnp.concatenate([pts, tt], axis=-1).reshape(-1, 3)
    coords = jnp.clip(coords, 0.0, 1.0)
    resolutions, sizes, offsets = _levels()
    emb = embeddings[:, 0]
    z = jnp.zeros((coords.shape[0],), jnp.float32)
    for l in range(NUM_LEVELS):
        res = resolutions[l]
        size = sizes[l]
        off = offsets[l]
        pos = coords * res
        pos0 = jnp.clip(jnp.floor(pos).astype(jnp.int32), 0, res - 1)
        frac = pos - pos0.astype(jnp.float32)
        feat = jnp.zeros((coords.shape[0],), jnp.float32)
        for c in range(8):
            dx, dy, dz = c & 1, (c >> 1) & 1, (c >> 2) & 1
            corner = jnp.clip(pos0 + jnp.array([dx, dy, dz], jnp.int32)[None, :], 0, res)
            wx = frac[:, 0] if dx else 1.0 - frac[:, 0]
            wy = frac[:, 1] if dy else 1.0 - frac[:, 1]
            wz = frac[:, 2] if dz else 1.0 - frac[:, 2]
            w = wx * wy * wz
            if (res + 1) ** 3 <= HASHMAP_SIZE:
                idx = corner[:, 0] + corner[:, 1] * (res + 1) + corner[:, 2] * (res + 1) ** 2
            else:
                cu = corner.astype(jnp.uint32)
                h = (cu[:, 0] * jnp.uint32(PRIMES[0])) ^ (cu[:, 1] * jnp.uint32(PRIMES[1])) ^ (cu[:, 2] * jnp.uint32(PRIMES[2]))
                idx = (h % jnp.uint32(size)).astype(jnp.int32)
            feat = feat + w * jnp.take(emb, off + idx, axis=0)
        z = z + W[0, l] * feat
    return (z + b[0]).reshape(N, N_POINTS)


def _tail_kernel(z_ref, out_ref, hits_ref, idx_ref):
    z = z_ref[...]
    s = jax.nn.sigmoid(z)
    out_ref[...] = s
    hits_ref[...] = jnp.max(s, axis=1, keepdims=True)
    labels = s > 0.5
    iota = jax.lax.broadcasted_iota(jnp.int32, z.shape, 1)
    first = jnp.min(jnp.where(labels, iota, N_POINTS), axis=1, keepdims=True)
    idx_ref[...] = jnp.where(first == N_POINTS, 0, first)


def kernel(x, embeddings, W, b):
    N = x.shape[0]
    z = _encode_z(x, embeddings, W, b)
    out, hits, idx = pl.pallas_call(
        _tail_kernel,
        out_shape=(
            jax.ShapeDtypeStruct((N, N_POINTS), jnp.float32),
            jax.ShapeDtypeStruct((N, 1), jnp.float32),
            jax.ShapeDtypeStruct((N, 1), jnp.int32),
        ),
    )(z)
    return (hits, out.reshape(N, N_POINTS, 1), idx.reshape(-1))


# speedup vs baseline: 13.3938x; 13.1254x over previous
import jax
import jax.numpy as jnp
from jax import lax
from jax.experimental import pallas as pl
from jax.experimental.pallas import tpu as pltpu
from jax.experimental.pallas import tpu_sc as plsc

N_RAYS = 4096
N_POINTS = 200
NV = 13
NPAD = NV * 16
B_R = 8
RAYS_PER_W = N_RAYS // 32
N_CHUNKS = RAYS_PER_W // B_R
RL = NV * 128

RES = (8, 16, 32, 64, 128)
OFFS = (0, 729, 5642, 41579, 316204)
TBL012 = 41584
P1 = 2654435761
P2 = 805459861
HASH_MASK = (1 << 19) - 1
BIGI = 1 << 30


def _sc_body(x_hbm, emb_hbm, t_hbm, wb_hbm,
             out_hbm, hits_hbm, idxf_hbm,
             tbl_v, xv, tv, wbv, idx3, idx4, w3, w4, val3, val4,
             accb, hitsb, idxfb, gsem, osem):
    wid = lax.axis_index("s") * 2 + lax.axis_index("c")
    base_ray = wid * RAYS_PER_W
    pltpu.sync_copy(emb_hbm.at[pl.ds(0, TBL012)], tbl_v)
    pltpu.sync_copy(x_hbm.at[pl.ds(base_ray * 4, RAYS_PER_W * 4)],
                    xv.at[pl.ds(0, RAYS_PER_W * 4)])
    pltpu.sync_copy(t_hbm, tv)
    pltpu.sync_copy(wb_hbm, wbv)
    iota = lax.iota(jnp.int32, 16)
    lane0 = iota == 0
    wv = wbv[pl.ds(0, 16)]
    wl = [wv[l] for l in range(5)]
    bias = wv[5]

    def pass_a(r, j, ray_l):
        row = xv[pl.ds(ray_l * 4, 16)]
        p0x = row[0]
        p0y = row[1]
        p1x = row[2]
        p1y = row[3]
        t = tv[pl.ds(j * 16, 16)]
        cx = jnp.clip(p0x + t * (p1x - p0x), 0.0, 1.0)
        cy = jnp.clip(p0y + t * (p1y - p0y), 0.0, 1.0)
        cz = jnp.clip(t, 0.0, 1.0)
        acc = jnp.zeros((16,), jnp.float32)
        for l in range(5):
            res = RES[l]
            px = cx * float(res)
            py = cy * float(res)
            pz = cz * float(res)
            x0 = jnp.clip(px.astype(jnp.int32), 0, res - 1)
            y0 = jnp.clip(py.astype(jnp.int32), 0, res - 1)
            z0 = jnp.clip(pz.astype(jnp.int32), 0, res - 1)
            fx = px - x0.astype(jnp.float32)
            fy = py - y0.astype(jnp.float32)
            fz = pz - z0.astype(jnp.float32)
            xs = (x0, jnp.minimum(x0 + 1, res))
            ys = (y0, jnp.minimum(y0 + 1, res))
            zs = (z0, jnp.minimum(z0 + 1, res))
            wx = (1.0 - fx, fx)
            wy = (1.0 - fy, fy)
            wz = (1.0 - fz, fz)
            if l < 4:
                m = res + 1
                ym = (ys[0] * m, ys[1] * m)
                zm = (OFFS[l] + zs[0] * (m * m), OFFS[l] + zs[1] * (m * m))
            if l < 3:
                feat = jnp.zeros((16,), jnp.float32)
                for c in range(8):
                    dx, dy, dz = c & 1, (c >> 1) & 1, (c >> 2) & 1
                    w = (wx[dx] * wy[dy]) * wz[dz]
                    idxv = xs[dx] + ym[dy] + zm[dz]
                    v = plsc.load_gather(tbl_v, [idxv])
                    feat = feat + w * v
                acc = acc + wl[l] * feat
            elif l == 3:
                for c in range(8):
                    dx, dy, dz = c & 1, (c >> 1) & 1, (c >> 2) & 1
                    w = (wx[dx] * wy[dy]) * wz[dz]
                    o = r * RL + j * 128 + c * 16
                    idx3[pl.ds(o, 16)] = xs[dx] + ym[dy] + zm[dz]
                    w3[pl.ds(o, 16)] = w
            else:
                xu = (xs[0].astype(jnp.uint32), xs[1].astype(jnp.uint32))
                yu = (ys[0].astype(jnp.uint32) * jnp.uint32(P1),
                      ys[1].astype(jnp.uint32) * jnp.uint32(P1))
                zu = (zs[0].astype(jnp.uint32) * jnp.uint32(P2),
                      zs[1].astype(jnp.uint32) * jnp.uint32(P2))
                for c in range(8):
                    dx, dy, dz = c & 1, (c >> 1) & 1, (c >> 2) & 1
                    w = (wx[dx] * wy[dy]) * wz[dz]
                    h = (xu[dx] ^ yu[dy] ^ zu[dz]) & jnp.uint32(HASH_MASK)
                    o = r * RL + j * 128 + c * 16
                    idx4[pl.ds(o, 16)] = h.astype(jnp.int32) + OFFS[4]
                    w4[pl.ds(o, 16)] = w
        accb[pl.ds(r * NPAD + j * 16, 16)] = acc

    def chunk_body(ch, _):
        cb_l = ch * B_R

        def ray_a(r, _2):
            def vec_a(j, _3):
                pass_a(r, j, cb_l + r)
                return 0
            lax.fori_loop(0, NV, vec_a, 0)
            return 0
        lax.fori_loop(0, B_R, ray_a, 0)

        descs = []
        for r in range(B_R):
            descs.append(pltpu.async_copy(
                emb_hbm.at[idx3.at[pl.ds(r * RL, RL)]],
                val3.at[pl.ds(r * RL, RL)], gsem))
            descs.append(pltpu.async_copy(
                emb_hbm.at[idx4.at[pl.ds(r * RL, RL)]],
                val4.at[pl.ds(r * RL, RL)], gsem))
        for d in descs:
            d.wait()

        def ray_b(r, _2):
            ray_l = cb_l + r

            def vec_b(j, carry):
                vmax, vmin = carry
                zv = accb[pl.ds(r * NPAD + j * 16, 16)]
                f3 = jnp.zeros((16,), jnp.float32)
                f4 = jnp.zeros((16,), jnp.float32)
                for c in range(8):
                    o = r * RL + j * 128 + c * 16
                    f3 = f3 + w3[pl.ds(o, 16)] * val3[pl.ds(o, 16)]
                for c in range(8):
                    o = r * RL + j * 128 + c * 16
                    f4 = f4 + w4[pl.ds(o, 16)] * val4[pl.ds(o, 16)]
                zv = zv + wl[3] * f3
                zv = zv + wl[4] * f4
                zv = zv + bias
                s = 1.0 / (1.0 + jnp.exp(-zv))
                accb[pl.ds(r * NPAD + j * 16, 16)] = s
                vmax = jnp.maximum(vmax, s)
                vmin = jnp.minimum(vmin, jnp.where(s > 0.5, j * 16 + iota, BIGI))
                return (vmax, vmin)

            vmax, vmin = lax.fori_loop(
                0, NV, vec_b,
                (jnp.zeros((16,), jnp.float32), jnp.full((16,), BIGI, jnp.int32)))
            mi = jnp.min(vmin)
            ray_i = jnp.full((16,), ray_l, jnp.int32)
            plsc.store_scatter(hitsb, [ray_i],
                               jnp.full((16,), jnp.max(vmax), jnp.float32), mask=lane0)
            first = jnp.where(mi >= BIGI, 0, mi)
            plsc.store_scatter(idxfb, [ray_i],
                               jnp.full((16,), first, jnp.int32), mask=lane0)
            return 0
        lax.fori_loop(0, B_R, ray_b, 0)

        odescs = []
        for r in range(B_R):
            ray_g = base_ray + cb_l + r
            odescs.append(pltpu.async_copy(
                accb.at[pl.ds(r * NPAD, N_POINTS)],
                out_hbm.at[pl.ds(ray_g * N_POINTS, N_POINTS)], osem))
        for d in odescs:
            d.wait()
        return 0

    lax.fori_loop(0, N_CHUNKS, chunk_body, 0)
    pltpu.sync_copy(hitsb, hits_hbm.at[pl.ds(base_ray, RAYS_PER_W)])
    pltpu.sync_copy(idxfb, idxf_hbm.at[pl.ds(base_ray, RAYS_PER_W)])


def _make_fwd():
    mesh = plsc.VectorSubcoreMesh(core_axis_name="c", subcore_axis_name="s")
    return pl.kernel(
        _sc_body,
        out_type=(
            jax.ShapeDtypeStruct((N_RAYS * N_POINTS,), jnp.float32),
            jax.ShapeDtypeStruct((N_RAYS,), jnp.float32),
            jax.ShapeDtypeStruct((N_RAYS,), jnp.int32),
        ),
        mesh=mesh,
        compiler_params=pltpu.CompilerParams(needs_layout_passes=False),
        scratch_types=[
            pltpu.VMEM((TBL012,), jnp.float32),
            pltpu.VMEM((RAYS_PER_W * 4 + 16,), jnp.float32),
            pltpu.VMEM((NPAD,), jnp.float32),
            pltpu.VMEM((16,), jnp.float32),
            pltpu.VMEM((B_R * RL,), jnp.int32),
            pltpu.VMEM((B_R * RL,), jnp.int32),
            pltpu.VMEM((B_R * RL,), jnp.float32),
            pltpu.VMEM((B_R * RL,), jnp.float32),
            pltpu.VMEM((B_R * RL,), jnp.float32),
            pltpu.VMEM((B_R * RL,), jnp.float32),
            pltpu.VMEM((B_R * NPAD,), jnp.float32),
            pltpu.VMEM((RAYS_PER_W,), jnp.float32),
            pltpu.VMEM((RAYS_PER_W,), jnp.int32),
            pltpu.SemaphoreType.DMA,
            pltpu.SemaphoreType.DMA,
        ],
    )


def kernel(x, embeddings, W, b):
    emb = embeddings.reshape(-1)
    t_all = jnp.linspace(0.0, 1.0, N_POINTS, dtype=jnp.float32)
    t_pad = jnp.concatenate([t_all, jnp.broadcast_to(t_all[-1], (NPAD - N_POINTS,))])
    wb = jnp.concatenate([W[0], b, jnp.zeros((10,), jnp.float32)])
    out2d, hits, idxf = _make_fwd()(x.reshape(-1), emb, t_pad, wb)
    return (hits.reshape(N_RAYS, 1), out2d.reshape(N_RAYS, N_POINTS, 1), idxf)


# speedup vs baseline: 18.2948x; 1.3659x over previous
import jax
import jax.numpy as jnp
from jax import lax
from jax.experimental import pallas as pl
from jax.experimental.pallas import tpu as pltpu
from jax.experimental.pallas import tpu_sc as plsc

N_RAYS = 4096
N_POINTS = 200
NV = 13
NPAD = NV * 16
RAYS_PER_W = N_RAYS // 32
RL = NV * 128
RING = 4

RES = (8, 16, 32, 64, 128)
OFFS = (0, 729, 5642, 41579, 316204)
TBL012 = 41584
P1 = 2654435761
P2 = 805459861
HASH_MASK = (1 << 19) - 1
BIGI = 1 << 30
OUTW = RAYS_PER_W * N_POINTS


def _sc_body(x_hbm, emb_hbm, t_hbm, wb_hbm,
             out_hbm, hits_hbm, idxf_hbm,
             tbl_v, xv, tv, wbv, zmv, zuv, fzv,
             idx3, idx4, w3, w4, val3, val4,
             accb, hitsb, idxfb, gsem):
    wid = lax.axis_index("s") * 2 + lax.axis_index("c")
    base_ray = wid * RAYS_PER_W
    pltpu.sync_copy(emb_hbm.at[pl.ds(0, TBL012)], tbl_v)
    pltpu.sync_copy(x_hbm.at[pl.ds(base_ray * 4, RAYS_PER_W * 4)],
                    xv.at[pl.ds(0, RAYS_PER_W * 4)])
    pltpu.sync_copy(t_hbm, tv)
    pltpu.sync_copy(wb_hbm, wbv)
    iota = lax.iota(jnp.int32, 16)
    lane0 = iota == 0
    mask8 = iota < 8
    wv = wbv[pl.ds(0, 16)]
    wl = [wv[l] for l in range(5)]
    bias = wv[5]

    def zprep(j, _):
        t = tv[pl.ds(j * 16, 16)]
        cz = jnp.clip(t, 0.0, 1.0)
        for l in range(5):
            res = RES[l]
            pz = cz * float(res)
            z0 = jnp.minimum(pz.astype(jnp.int32), res - 1)
            fz = pz - z0.astype(jnp.float32)
            z1 = jnp.minimum(z0 + 1, res)
            fzv[pl.ds(l * NPAD + j * 16, 16)] = fz
            if l < 4:
                m2 = (res + 1) * (res + 1)
                zmv[pl.ds((2 * l) * NPAD + j * 16, 16)] = OFFS[l] + z0 * m2
                zmv[pl.ds((2 * l + 1) * NPAD + j * 16, 16)] = OFFS[l] + z1 * m2
            else:
                zuv[pl.ds(j * 16, 16)] = plsc.bitcast(
                    z0.astype(jnp.uint32) * jnp.uint32(P2), jnp.int32)
                zuv[pl.ds(NPAD + j * 16, 16)] = plsc.bitcast(
                    z1.astype(jnp.uint32) * jnp.uint32(P2), jnp.int32)
        return 0
    lax.fori_loop(0, NV, zprep, 0)

    def pass_a_vec(ra, slot, j):
        row = xv[pl.ds(ra * 4, 16)]
        p0x = row[0]
        p0y = row[1]
        p1x = row[2]
        p1y = row[3]
        t = tv[pl.ds(j * 16, 16)]
        cx = jnp.clip(p0x + t * (p1x - p0x), 0.0, 1.0)
        cy = jnp.clip(p0y + t * (p1y - p0y), 0.0, 1.0)
        acc = jnp.zeros((16,), jnp.float32)
        for l in range(5):
            res = RES[l]
            px = cx * float(res)
            py = cy * float(res)
            x0 = jnp.minimum(px.astype(jnp.int32), res - 1)
            y0 = jnp.minimum(py.astype(jnp.int32), res - 1)
            fx = px - x0.astype(jnp.float32)
            fy = py - y0.astype(jnp.float32)
            xs = (x0, jnp.minimum(x0 + 1, res))
            ys = (y0, jnp.minimum(y0 + 1, res))
            fz = fzv[pl.ds(l * NPAD + j * 16, 16)]
            wx = (1.0 - fx, fx)
            wy = (1.0 - fy, fy)
            wz = (1.0 - fz, fz)
            wxy = (wx[0] * wy[0], wx[1] * wy[0], wx[0] * wy[1], wx[1] * wy[1])
            if l < 4:
                m = res + 1
                ym = (ys[0] * m, ys[1] * m)
                zm = (zmv[pl.ds((2 * l) * NPAD + j * 16, 16)],
                      zmv[pl.ds((2 * l + 1) * NPAD + j * 16, 16)])
            if l < 3:
                feat = jnp.zeros((16,), jnp.float32)
                for c in range(8):
                    dx, dy, dz = c & 1, (c >> 1) & 1, (c >> 2) & 1
                    w = wxy[dy * 2 + dx] * wz[dz]
                    idxv = xs[dx] + ym[dy] + zm[dz]
                    v = plsc.load_gather(tbl_v, [idxv])
                    feat = feat + w * v
                acc = acc + wl[l] * feat
            elif l == 3:
                xy = (xs[0] + ym[0], xs[1] + ym[0], xs[0] + ym[1], xs[1] + ym[1])
                for c in range(8):
                    dx, dy, dz = c & 1, (c >> 1) & 1, (c >> 2) & 1
                    o = slot * RL + j * 128 + c * 16
                    idx3[pl.ds(o, 16)] = xy[dy * 2 + dx] + zm[dz]
                    w3[pl.ds(o, 16)] = wxy[dy * 2 + dx] * wz[dz]
            else:
                xu = (xs[0].astype(jnp.uint32), xs[1].astype(jnp.uint32))
                yu = (ys[0].astype(jnp.uint32) * jnp.uint32(P1),
                      ys[1].astype(jnp.uint32) * jnp.uint32(P1))
                xyu = (xu[0] ^ yu[0], xu[1] ^ yu[0], xu[0] ^ yu[1], xu[1] ^ yu[1])
                zu = (plsc.bitcast(zuv[pl.ds(j * 16, 16)], jnp.uint32),
                      plsc.bitcast(zuv[pl.ds(NPAD + j * 16, 16)], jnp.uint32))
                for c in range(8):
                    dx, dy, dz = c & 1, (c >> 1) & 1, (c >> 2) & 1
                    h = (xyu[dy * 2 + dx] ^ zu[dz]) & jnp.uint32(HASH_MASK)
                    o = slot * RL + j * 128 + c * 16
                    idx4[pl.ds(o, 16)] = h.astype(jnp.int32) + OFFS[4]
                    w4[pl.ds(o, 16)] = wxy[dy * 2 + dx] * wz[dz]
        accb[pl.ds(ra * N_POINTS + j * 16, 16)] = acc

    def run_pass_a(ra, slot):
        def vec_a(j, _):
            pass_a_vec(ra, slot, j)
            return 0
        lax.fori_loop(0, NV, vec_a, 0)

    def fire(slot):
        pltpu.async_copy(emb_hbm.at[idx3.at[pl.ds(slot * RL, RL)]],
                         val3.at[pl.ds(slot * RL, RL)], gsem.at[slot])
        pltpu.async_copy(emb_hbm.at[idx4.at[pl.ds(slot * RL, RL)]],
                         val4.at[pl.ds(slot * RL, RL)], gsem.at[slot])

    def wait_pair(slot):
        pltpu.make_async_copy(emb_hbm.at[idx3.at[pl.ds(slot * RL, RL)]],
                              val3.at[pl.ds(slot * RL, RL)], gsem.at[slot]).wait()
        pltpu.make_async_copy(emb_hbm.at[idx4.at[pl.ds(slot * RL, RL)]],
                              val4.at[pl.ds(slot * RL, RL)], gsem.at[slot]).wait()

    def pass_b_vec(r, slot, j, tail):
        zv = accb[pl.ds(r * N_POINTS + j * 16, 16)]
        f3 = jnp.zeros((16,), jnp.float32)
        f4 = jnp.zeros((16,), jnp.float32)
        for c in range(8):
            o = slot * RL + j * 128 + c * 16
            f3 = f3 + w3[pl.ds(o, 16)] * val3[pl.ds(o, 16)]
        for c in range(8):
            o = slot * RL + j * 128 + c * 16
            f4 = f4 + w4[pl.ds(o, 16)] * val4[pl.ds(o, 16)]
        zv = zv + wl[3] * f3
        zv = zv + wl[4] * f4
        zv = zv + bias
        s = 1.0 / (1.0 + jnp.exp(-zv))
        hit = s > 0.5
        pidx = j * 16 + iota
        if tail:
            plsc.store_compressed(accb.at[pl.ds(r * N_POINTS + j * 16, 16)], s, mask=mask8)
            smax = jnp.where(mask8, s, 0.0)
            cand = jnp.where(hit & mask8, pidx, BIGI)
        else:
            accb[pl.ds(r * N_POINTS + j * 16, 16)] = s
            smax = s
            cand = jnp.where(hit, pidx, BIGI)
        return smax, cand

    def run_pass_b(r, slot):
        def vec_b(j, carry):
            vmax, vmin = carry
            smax, cand = pass_b_vec(r, slot, j, False)
            return (jnp.maximum(vmax, smax), jnp.minimum(vmin, cand))
        vmax, vmin = lax.fori_loop(
            0, NV - 1, vec_b,
            (jnp.zeros((16,), jnp.float32), jnp.full((16,), BIGI, jnp.int32)))
        smax, cand = pass_b_vec(r, slot, NV - 1, True)
        vmax = jnp.maximum(vmax, smax)
        vmin = jnp.minimum(vmin, cand)
        mi = jnp.min(vmin)
        ray_i = jnp.full((16,), r, jnp.int32)
        plsc.store_scatter(hitsb, [ray_i],
                           jnp.full((16,), jnp.max(vmax), jnp.float32), mask=lane0)
        first = jnp.where(mi >= BIGI, 0, mi)
        plsc.store_scatter(idxfb, [ray_i],
                           jnp.full((16,), first, jnp.int32), mask=lane0)

    for k in range(RING - 1):
        run_pass_a(k, k)
        fire(k)

    NQ = RAYS_PER_W // RING
    def quad_iter(q, _):
        r0 = q * RING
        for k in range(RING):
            kn = (k + RING - 1) % RING
            wait_pair(k)
            run_pass_b(r0 + k, k)
            run_pass_a(r0 + k + RING - 1, kn)
            fire(kn)
        return 0
    lax.fori_loop(0, NQ - 1, quad_iter, 0)

    run_pass_a(RAYS_PER_W - 1, (RING - 1) % RING)
    fire(RING - 1)
    for k in range(RING):
        wait_pair(k)
        run_pass_b(RAYS_PER_W - RING + k, k)

    pltpu.sync_copy(accb.at[pl.ds(0, OUTW)],
                    out_hbm.at[pl.ds(base_ray * N_POINTS, OUTW)])
    pltpu.sync_copy(hitsb, hits_hbm.at[pl.ds(base_ray, RAYS_PER_W)])
    pltpu.sync_copy(idxfb, idxf_hbm.at[pl.ds(base_ray, RAYS_PER_W)])


def _make_fwd():
    mesh = plsc.VectorSubcoreMesh(core_axis_name="c", subcore_axis_name="s")
    return pl.kernel(
        _sc_body,
        out_type=(
            jax.ShapeDtypeStruct((N_RAYS * N_POINTS,), jnp.float32),
            jax.ShapeDtypeStruct((N_RAYS,), jnp.float32),
            jax.ShapeDtypeStruct((N_RAYS,), jnp.int32),
        ),
        mesh=mesh,
        compiler_params=pltpu.CompilerParams(needs_layout_passes=False),
        scratch_types=[
            pltpu.VMEM((TBL012,), jnp.float32),
            pltpu.VMEM((RAYS_PER_W * 4 + 16,), jnp.float32),
            pltpu.VMEM((NPAD,), jnp.float32),
            pltpu.VMEM((16,), jnp.float32),
            pltpu.VMEM((8 * NPAD,), jnp.int32),
            pltpu.VMEM((2 * NPAD,), jnp.int32),
            pltpu.VMEM((5 * NPAD,), jnp.float32),
            pltpu.VMEM((RING * RL,), jnp.int32),
            pltpu.VMEM((RING * RL,), jnp.int32),
            pltpu.VMEM((RING * RL,), jnp.float32),
            pltpu.VMEM((RING * RL,), jnp.float32),
            pltpu.VMEM((RING * RL,), jnp.float32),
            pltpu.VMEM((RING * RL,), jnp.float32),
            pltpu.VMEM((OUTW + 16,), jnp.float32),
            pltpu.VMEM((RAYS_PER_W,), jnp.float32),
            pltpu.VMEM((RAYS_PER_W,), jnp.int32),
            pltpu.SemaphoreType.DMA((RING,)),
        ],
    )


def kernel(x, embeddings, W, b):
    emb = embeddings.reshape(-1)
    t_all = jnp.linspace(0.0, 1.0, N_POINTS, dtype=jnp.float32)
    t_pad = jnp.concatenate([t_all, jnp.broadcast_to(t_all[-1], (NPAD - N_POINTS,))])
    wb = jnp.concatenate([W[0], b, jnp.zeros((10,), jnp.float32)])
    out1d, hits, idxf = _make_fwd()(x.reshape(-1), emb, t_pad, wb)
    return (hits.reshape(N_RAYS, 1), out1d.reshape(N_RAYS, N_POINTS, 1), idxf)


# speedup vs baseline: 29.3496x; 1.6043x over previous
import jax
import jax.numpy as jnp
from jax import lax
from jax.experimental import pallas as pl
from jax.experimental.pallas import tpu as pltpu
from jax.experimental.pallas import tpu_sc as plsc

N_RAYS = 4096
N_POINTS = 200
NV = 13
NPAD = NV * 16
RAYS_PER_W = N_RAYS // 32
RL = NV * 128
RING = 4

RES = (8, 16, 32, 64, 128)
OFFS = (0, 729, 5642, 41579, 316204)
TBL012 = 41584
P1 = 2654435761
P2 = 805459861
HASH_MASK = (1 << 19) - 1
BIGI = 1 << 30
OUTW = RAYS_PER_W * N_POINTS
SBASE = 41576
L3W = 274628
SOFF = (0, 729, 5642, 41579 - SBASE, 316204 - SBASE)


def _sc_body(x_hbm, emb_hbm, t_hbm, wb_hbm,
             out_hbm, hits_hbm, idxf_hbm,
             tbl_v, xv, tv, wbv, zmv, zuv, fzv,
             idx3, idx4, w3, w4, val3, val4,
             accb, hitsb, idxfb, spm, gsem):
    wid = lax.axis_index("s") * 2 + lax.axis_index("c")
    base_ray = wid * RAYS_PER_W
    pltpu.sync_copy(emb_hbm.at[pl.ds(0, TBL012)], tbl_v)
    pltpu.sync_copy(x_hbm.at[pl.ds(base_ray * 4, RAYS_PER_W * 4)],
                    xv.at[pl.ds(0, RAYS_PER_W * 4)])
    pltpu.sync_copy(t_hbm, tv)
    pltpu.sync_copy(wb_hbm, wbv)
    sid = lax.axis_index("s")
    soff0 = sid * 17168
    pltpu.sync_copy(emb_hbm.at[pl.ds(SBASE + soff0, 17168)], accb.at[pl.ds(0, 17168)])
    pltpu.sync_copy(accb.at[pl.ds(0, 17168)], spm.at[pl.ds(soff0, 17168)])
    plsc.subcore_barrier()
    iota = lax.iota(jnp.int32, 16)
    lane0 = iota == 0
    mask8 = iota < 8
    wv = wbv[pl.ds(0, 16)]
    wl = [wv[l] for l in range(5)]
    bias = wv[5]

    def zprep(j, _):
        t = tv[pl.ds(j * 16, 16)]
        cz = jnp.clip(t, 0.0, 1.0)
        for l in range(5):
            res = RES[l]
            pz = cz * float(res)
            z0 = jnp.minimum(pz.astype(jnp.int32), res - 1)
            fz = pz - z0.astype(jnp.float32)
            z1 = jnp.minimum(z0 + 1, res)
            fzv[pl.ds(l * NPAD + j * 16, 16)] = fz
            if l < 4:
                m2 = (res + 1) * (res + 1)
                zmv[pl.ds((2 * l) * NPAD + j * 16, 16)] = SOFF[l] + z0 * m2
                zmv[pl.ds((2 * l + 1) * NPAD + j * 16, 16)] = SOFF[l] + z1 * m2
            else:
                zuv[pl.ds(j * 16, 16)] = plsc.bitcast(
                    z0.astype(jnp.uint32) * jnp.uint32(P2), jnp.int32)
                zuv[pl.ds(NPAD + j * 16, 16)] = plsc.bitcast(
                    z1.astype(jnp.uint32) * jnp.uint32(P2), jnp.int32)
        return 0
    lax.fori_loop(0, NV, zprep, 0)

    def pass_a_vec(ra, slot, j):
        row = xv[pl.ds(ra * 4, 16)]
        p0x = row[0]
        p0y = row[1]
        p1x = row[2]
        p1y = row[3]
        t = tv[pl.ds(j * 16, 16)]
        cx = jnp.clip(p0x + t * (p1x - p0x), 0.0, 1.0)
        cy = jnp.clip(p0y + t * (p1y - p0y), 0.0, 1.0)
        acc = jnp.zeros((16,), jnp.float32)
        for l in range(5):
            res = RES[l]
            px = cx * float(res)
            py = cy * float(res)
            x0 = jnp.minimum(px.astype(jnp.int32), res - 1)
            y0 = jnp.minimum(py.astype(jnp.int32), res - 1)
            fx = px - x0.astype(jnp.float32)
            fy = py - y0.astype(jnp.float32)
            xs = (x0, jnp.minimum(x0 + 1, res))
            ys = (y0, jnp.minimum(y0 + 1, res))
            fz = fzv[pl.ds(l * NPAD + j * 16, 16)]
            wx = (1.0 - fx, fx)
            wy = (1.0 - fy, fy)
            wz = (1.0 - fz, fz)
            wxy = (wx[0] * wy[0], wx[1] * wy[0], wx[0] * wy[1], wx[1] * wy[1])
            if l < 4:
                m = res + 1
                ym = (ys[0] * m, ys[1] * m)
                zm = (zmv[pl.ds((2 * l) * NPAD + j * 16, 16)],
                      zmv[pl.ds((2 * l + 1) * NPAD + j * 16, 16)])
            if l < 3:
                feat = jnp.zeros((16,), jnp.float32)
                for c in range(8):
                    dx, dy, dz = c & 1, (c >> 1) & 1, (c >> 2) & 1
                    w = wxy[dy * 2 + dx] * wz[dz]
                    idxv = xs[dx] + ym[dy] + zm[dz]
                    v = plsc.load_gather(tbl_v, [idxv])
                    feat = feat + w * v
                acc = acc + wl[l] * feat
            elif l == 3:
                xy = (xs[0] + ym[0], xs[1] + ym[0], xs[0] + ym[1], xs[1] + ym[1])
                for c in range(8):
                    dx, dy, dz = c & 1, (c >> 1) & 1, (c >> 2) & 1
                    o = slot * RL + j * 128 + c * 16
                    idx3[pl.ds(o, 16)] = xy[dy * 2 + dx] + zm[dz]
                    w3[pl.ds(o, 16)] = wxy[dy * 2 + dx] * wz[dz]
            else:
                xu = (xs[0].astype(jnp.uint32), xs[1].astype(jnp.uint32))
                yu = (ys[0].astype(jnp.uint32) * jnp.uint32(P1),
                      ys[1].astype(jnp.uint32) * jnp.uint32(P1))
                xyu = (xu[0] ^ yu[0], xu[1] ^ yu[0], xu[0] ^ yu[1], xu[1] ^ yu[1])
                zu = (plsc.bitcast(zuv[pl.ds(j * 16, 16)], jnp.uint32),
                      plsc.bitcast(zuv[pl.ds(NPAD + j * 16, 16)], jnp.uint32))
                for c in range(8):
                    dx, dy, dz = c & 1, (c >> 1) & 1, (c >> 2) & 1
                    h = (xyu[dy * 2 + dx] ^ zu[dz]) & jnp.uint32(HASH_MASK)
                    o = slot * RL + j * 128 + c * 16
                    idx4[pl.ds(o, 16)] = h.astype(jnp.int32) + OFFS[4]
                    w4[pl.ds(o, 16)] = wxy[dy * 2 + dx] * wz[dz]
        accb[pl.ds(ra * N_POINTS + j * 16, 16)] = acc

    def run_pass_a(ra, slot):
        def vec_a(j, _):
            pass_a_vec(ra, slot, j)
            return 0
        lax.fori_loop(0, NV, vec_a, 0)

    def fire(slot):
        pltpu.async_copy(emb_hbm.at[idx4.at[pl.ds(slot * RL, RL)]],
                         val4.at[pl.ds(slot * RL, RL)], gsem.at[slot])

    def wait_pair(slot):
        pltpu.sync_copy(spm.at[idx3.at[pl.ds(slot * RL, RL)]],
                        val3.at[pl.ds(slot * RL, RL)])
        pltpu.make_async_copy(emb_hbm.at[idx4.at[pl.ds(slot * RL, RL)]],
                              val4.at[pl.ds(slot * RL, RL)], gsem.at[slot]).wait()

    def pass_b_vec(r, slot, j, tail):
        zv = accb[pl.ds(r * N_POINTS + j * 16, 16)]
        f3 = jnp.zeros((16,), jnp.float32)
        f4 = jnp.zeros((16,), jnp.float32)
        for c in range(8):
            o = slot * RL + j * 128 + c * 16
            f3 = f3 + w3[pl.ds(o, 16)] * val3[pl.ds(o, 16)]
        for c in range(8):
            o = slot * RL + j * 128 + c * 16
            f4 = f4 + w4[pl.ds(o, 16)] * val4[pl.ds(o, 16)]
        zv = zv + wl[3] * f3
        zv = zv + wl[4] * f4
        zv = zv + bias
        s = 1.0 / (1.0 + jnp.exp(-zv))
        hit = s > 0.5
        pidx = j * 16 + iota
        if tail:
            plsc.store_compressed(accb.at[pl.ds(r * N_POINTS + j * 16, 16)], s, mask=mask8)
            smax = jnp.where(mask8, s, 0.0)
            cand = jnp.where(hit & mask8, pidx, BIGI)
        else:
            accb[pl.ds(r * N_POINTS + j * 16, 16)] = s
            smax = s
            cand = jnp.where(hit, pidx, BIGI)
        return smax, cand

    def run_pass_b(r, slot):
        def vec_b(j, carry):
            vmax, vmin = carry
            smax, cand = pass_b_vec(r, slot, j, False)
            return (jnp.maximum(vmax, smax), jnp.minimum(vmin, cand))
        vmax, vmin = lax.fori_loop(
            0, NV - 1, vec_b,
            (jnp.zeros((16,), jnp.float32), jnp.full((16,), BIGI, jnp.int32)))
        smax, cand = pass_b_vec(r, slot, NV - 1, True)
        vmax = jnp.maximum(vmax, smax)
        vmin = jnp.minimum(vmin, cand)
        mi = jnp.min(vmin)
        ray_i = jnp.full((16,), r, jnp.int32)
        plsc.store_scatter(hitsb, [ray_i],
                           jnp.full((16,), jnp.max(vmax), jnp.float32), mask=lane0)
        first = jnp.where(mi >= BIGI, 0, mi)
        plsc.store_scatter(idxfb, [ray_i],
                           jnp.full((16,), first, jnp.int32), mask=lane0)

    for k in range(RING - 1):
        run_pass_a(k, k)
        fire(k)

    NQ = RAYS_PER_W // RING
    def quad_iter(q, _):
        r0 = q * RING
        for k in range(RING):
            kn = (k + RING - 1) % RING
            wait_pair(k)
            run_pass_b(r0 + k, k)
            run_pass_a(r0 + k + RING - 1, kn)
            fire(kn)
        return 0
    lax.fori_loop(0, NQ - 1, quad_iter, 0)

    run_pass_a(RAYS_PER_W - 1, (RING - 1) % RING)
    fire(RING - 1)
    for k in range(RING):
        wait_pair(k)
        run_pass_b(RAYS_PER_W - RING + k, k)

    pltpu.sync_copy(accb.at[pl.ds(0, OUTW)],
                    out_hbm.at[pl.ds(base_ray * N_POINTS, OUTW)])
    pltpu.sync_copy(hitsb, hits_hbm.at[pl.ds(base_ray, RAYS_PER_W)])
    pltpu.sync_copy(idxfb, idxf_hbm.at[pl.ds(base_ray, RAYS_PER_W)])


def _make_fwd():
    mesh = plsc.VectorSubcoreMesh(core_axis_name="c", subcore_axis_name="s")
    return pl.kernel(
        _sc_body,
        out_type=(
            jax.ShapeDtypeStruct((N_RAYS * N_POINTS,), jnp.float32),
            jax.ShapeDtypeStruct((N_RAYS,), jnp.float32),
            jax.ShapeDtypeStruct((N_RAYS,), jnp.int32),
        ),
        mesh=mesh,
        compiler_params=pltpu.CompilerParams(needs_layout_passes=False),
        scratch_types=[
            pltpu.VMEM((TBL012,), jnp.float32),
            pltpu.VMEM((RAYS_PER_W * 4 + 16,), jnp.float32),
            pltpu.VMEM((NPAD,), jnp.float32),
            pltpu.VMEM((16,), jnp.float32),
            pltpu.VMEM((8 * NPAD,), jnp.int32),
            pltpu.VMEM((2 * NPAD,), jnp.int32),
            pltpu.VMEM((5 * NPAD,), jnp.float32),
            pltpu.VMEM((RING * RL,), jnp.int32),
            pltpu.VMEM((RING * RL,), jnp.int32),
            pltpu.VMEM((RING * RL,), jnp.float32),
            pltpu.VMEM((RING * RL,), jnp.float32),
            pltpu.VMEM((RING * RL,), jnp.float32),
            pltpu.VMEM((RING * RL,), jnp.float32),
            pltpu.VMEM((OUTW + 16,), jnp.float32),
            pltpu.VMEM((RAYS_PER_W,), jnp.float32),
            pltpu.VMEM((RAYS_PER_W,), jnp.int32),
            pltpu.VMEM_SHARED((16 * 17168,), jnp.float32),
            pltpu.SemaphoreType.DMA((RING,)),
        ],
    )


def kernel(x, embeddings, W, b):
    emb = embeddings.reshape(-1)
    t_all = jnp.linspace(0.0, 1.0, N_POINTS, dtype=jnp.float32)
    t_pad = jnp.concatenate([t_all, jnp.broadcast_to(t_all[-1], (NPAD - N_POINTS,))])
    wb = jnp.concatenate([W[0], b, jnp.zeros((10,), jnp.float32)])
    out1d, hits, idxf = _make_fwd()(x.reshape(-1), emb, t_pad, wb)
    return (hits.reshape(N_RAYS, 1), out1d.reshape(N_RAYS, N_POINTS, 1), idxf)


# speedup vs baseline: 32.7298x; 1.1152x over previous
import jax
import jax.numpy as jnp
from jax import lax
from jax.experimental import pallas as pl
from jax.experimental.pallas import tpu as pltpu
from jax.experimental.pallas import tpu_sc as plsc

N_RAYS = 4096
N_POINTS = 200
NV = 13
NPAD = NV * 16
RAYS_PER_W = N_RAYS // 32
RL = NV * 128
RING = 4

RES = (8, 16, 32, 64, 128)
OFFS = (0, 729, 5642, 41579, 316204)
TBL012 = 41584
P1 = 2654435761
P2 = 805459861
HASH_MASK = (1 << 19) - 1
BIGI = 1 << 30
OUTW = RAYS_PER_W * N_POINTS
SBASE = 41576
L3W = 274628
SOFF = (0, 729, 5642, 41579 - SBASE, 316204 - SBASE)


def _sc_body(x_hbm, emb_hbm, t_hbm, wb_hbm,
             out_hbm, hits_hbm, idxf_hbm,
             tbl_v, xv, tv, wbv, zmv, zuv, fzv,
             idx3, idx4, w3, w4, val3, val4,
             accb, hitsb, idxfb, spm, gsem, gsem3):
    wid = lax.axis_index("s") * 2 + lax.axis_index("c")
    base_ray = wid * RAYS_PER_W
    pltpu.sync_copy(emb_hbm.at[pl.ds(0, TBL012)], tbl_v)
    pltpu.sync_copy(x_hbm.at[pl.ds(base_ray * 4, RAYS_PER_W * 4)],
                    xv.at[pl.ds(0, RAYS_PER_W * 4)])
    pltpu.sync_copy(t_hbm, tv)
    pltpu.sync_copy(wb_hbm, wbv)
    sid = lax.axis_index("s")
    soff0 = sid * 17168
    pltpu.sync_copy(emb_hbm.at[pl.ds(SBASE + soff0, 17168)], accb.at[pl.ds(0, 17168)])
    pltpu.sync_copy(accb.at[pl.ds(0, 17168)], spm.at[pl.ds(soff0, 17168)])
    plsc.subcore_barrier()
    iota = lax.iota(jnp.int32, 16)
    lane0 = iota == 0
    mask8 = iota < 8
    wv = wbv[pl.ds(0, 16)]
    wl = [wv[l] for l in range(5)]
    bias = wv[5]

    def zprep(j, _):
        t = tv[pl.ds(j * 16, 16)]
        cz = jnp.clip(t, 0.0, 1.0)
        for l in range(5):
            res = RES[l]
            pz = cz * float(res)
            z0 = jnp.minimum(pz.astype(jnp.int32), res - 1)
            fz = pz - z0.astype(jnp.float32)
            z1 = jnp.minimum(z0 + 1, res)
            fzv[pl.ds(l * NPAD + j * 16, 16)] = fz
            if l < 4:
                m2 = (res + 1) * (res + 1)
                zmv[pl.ds((2 * l) * NPAD + j * 16, 16)] = SOFF[l] + z0 * m2
                zmv[pl.ds((2 * l + 1) * NPAD + j * 16, 16)] = SOFF[l] + z1 * m2
            else:
                zuv[pl.ds(j * 16, 16)] = plsc.bitcast(
                    z0.astype(jnp.uint32) * jnp.uint32(P2), jnp.int32)
                zuv[pl.ds(NPAD + j * 16, 16)] = plsc.bitcast(
                    z1.astype(jnp.uint32) * jnp.uint32(P2), jnp.int32)
        return 0
    lax.fori_loop(0, NV, zprep, 0)

    def pass_a_vec(ra, slot, j):
        row = xv[pl.ds(ra * 4, 16)]
        p0x = row[0]
        p0y = row[1]
        p1x = row[2]
        p1y = row[3]
        t = tv[pl.ds(j * 16, 16)]
        cx = jnp.clip(p0x + t * (p1x - p0x), 0.0, 1.0)
        cy = jnp.clip(p0y + t * (p1y - p0y), 0.0, 1.0)
        acc = jnp.zeros((16,), jnp.float32)
        for l in range(5):
            res = RES[l]
            px = cx * float(res)
            py = cy * float(res)
            x0 = jnp.minimum(px.astype(jnp.int32), res - 1)
            y0 = jnp.minimum(py.astype(jnp.int32), res - 1)
            fx = px - x0.astype(jnp.float32)
            fy = py - y0.astype(jnp.float32)
            xs = (x0, jnp.minimum(x0 + 1, res))
            ys = (y0, jnp.minimum(y0 + 1, res))
            fz = fzv[pl.ds(l * NPAD + j * 16, 16)]
            wx = (1.0 - fx, fx)
            wy = (1.0 - fy, fy)
            wz = (1.0 - fz, fz)
            wxy = (wx[0] * wy[0], wx[1] * wy[0], wx[0] * wy[1], wx[1] * wy[1])
            if l < 4:
                m = res + 1
                ym = (ys[0] * m, ys[1] * m)
                zm = (zmv[pl.ds((2 * l) * NPAD + j * 16, 16)],
                      zmv[pl.ds((2 * l + 1) * NPAD + j * 16, 16)])
            if l < 3:
                feat = jnp.zeros((16,), jnp.float32)
                for c in range(8):
                    dx, dy, dz = c & 1, (c >> 1) & 1, (c >> 2) & 1
                    w = wxy[dy * 2 + dx] * wz[dz]
                    idxv = xs[dx] + ym[dy] + zm[dz]
                    v = plsc.load_gather(tbl_v, [idxv])
                    feat = feat + w * v
                acc = acc + wl[l] * feat
            elif l == 3:
                xy = (xs[0] + ym[0], xs[1] + ym[0], xs[0] + ym[1], xs[1] + ym[1])
                for c in range(8):
                    dx, dy, dz = c & 1, (c >> 1) & 1, (c >> 2) & 1
                    o = slot * RL + j * 128 + c * 16
                    idx3[pl.ds(o, 16)] = xy[dy * 2 + dx] + zm[dz]
                    w3[pl.ds(o, 16)] = wxy[dy * 2 + dx] * wz[dz]
            else:
                xu = (xs[0].astype(jnp.uint32), xs[1].astype(jnp.uint32))
                yu = (ys[0].astype(jnp.uint32) * jnp.uint32(P1),
                      ys[1].astype(jnp.uint32) * jnp.uint32(P1))
                xyu = (xu[0] ^ yu[0], xu[1] ^ yu[0], xu[0] ^ yu[1], xu[1] ^ yu[1])
                zu = (plsc.bitcast(zuv[pl.ds(j * 16, 16)], jnp.uint32),
                      plsc.bitcast(zuv[pl.ds(NPAD + j * 16, 16)], jnp.uint32))
                for c in range(8):
                    dx, dy, dz = c & 1, (c >> 1) & 1, (c >> 2) & 1
                    h = (xyu[dy * 2 + dx] ^ zu[dz]) & jnp.uint32(HASH_MASK)
                    o = slot * RL + j * 128 + c * 16
                    idx4[pl.ds(o, 16)] = h.astype(jnp.int32) + OFFS[4]
                    w4[pl.ds(o, 16)] = wxy[dy * 2 + dx] * wz[dz]
        accb[pl.ds(ra * N_POINTS + j * 16, 16)] = acc

    def run_pass_a(ra, slot):
        def vec_a(j, _):
            pass_a_vec(ra, slot, j)
            return 0
        lax.fori_loop(0, NV, vec_a, 0)

    def fire(slot):
        pltpu.async_copy(spm.at[idx3.at[pl.ds(slot * RL, RL)]],
                         val3.at[pl.ds(slot * RL, RL)], gsem3.at[slot])
        pltpu.async_copy(emb_hbm.at[idx4.at[pl.ds(slot * RL, RL)]],
                         val4.at[pl.ds(slot * RL, RL)], gsem.at[slot])

    def wait_pair(slot):
        pltpu.make_async_copy(spm.at[idx3.at[pl.ds(slot * RL, RL)]],
                              val3.at[pl.ds(slot * RL, RL)], gsem3.at[slot]).wait()
        pltpu.make_async_copy(emb_hbm.at[idx4.at[pl.ds(slot * RL, RL)]],
                              val4.at[pl.ds(slot * RL, RL)], gsem.at[slot]).wait()

    def pass_b_vec(r, slot, j, tail):
        zv = accb[pl.ds(r * N_POINTS + j * 16, 16)]
        f3 = jnp.zeros((16,), jnp.float32)
        f4 = jnp.zeros((16,), jnp.float32)
        for c in range(8):
            o = slot * RL + j * 128 + c * 16
            f3 = f3 + w3[pl.ds(o, 16)] * val3[pl.ds(o, 16)]
        for c in range(8):
            o = slot * RL + j * 128 + c * 16
            f4 = f4 + w4[pl.ds(o, 16)] * val4[pl.ds(o, 16)]
        zv = zv + wl[3] * f3
        zv = zv + wl[4] * f4
        zv = zv + bias
        s = 1.0 / (1.0 + jnp.exp(-zv))
        hit = s > 0.5
        pidx = j * 16 + iota
        if tail:
            plsc.store_compressed(accb.at[pl.ds(r * N_POINTS + j * 16, 16)], s, mask=mask8)
            smax = jnp.where(mask8, s, 0.0)
            cand = jnp.where(hit & mask8, pidx, BIGI)
        else:
            accb[pl.ds(r * N_POINTS + j * 16, 16)] = s
            smax = s
            cand = jnp.where(hit, pidx, BIGI)
        return smax, cand

    def run_pass_b(r, slot):
        def vec_b(j, carry):
            vmax, vmin = carry
            smax, cand = pass_b_vec(r, slot, j, False)
            return (jnp.maximum(vmax, smax), jnp.minimum(vmin, cand))
        vmax, vmin = lax.fori_loop(
            0, NV - 1, vec_b,
            (jnp.zeros((16,), jnp.float32), jnp.full((16,), BIGI, jnp.int32)))
        smax, cand = pass_b_vec(r, slot, NV - 1, True)
        vmax = jnp.maximum(vmax, smax)
        vmin = jnp.minimum(vmin, cand)
        mi = jnp.min(vmin)
        ray_i = jnp.full((16,), r, jnp.int32)
        plsc.store_scatter(hitsb, [ray_i],
                           jnp.full((16,), jnp.max(vmax), jnp.float32), mask=lane0)
        first = jnp.where(mi >= BIGI, 0, mi)
        plsc.store_scatter(idxfb, [ray_i],
                           jnp.full((16,), first, jnp.int32), mask=lane0)

    for k in range(RING - 1):
        run_pass_a(k, k)
        fire(k)

    NQ = RAYS_PER_W // RING
    def quad_iter(q, _):
        r0 = q * RING
        for k in range(RING):
            kn = (k + RING - 1) % RING
            wait_pair(k)
            run_pass_b(r0 + k, k)
            run_pass_a(r0 + k + RING - 1, kn)
            fire(kn)
        return 0
    lax.fori_loop(0, NQ - 1, quad_iter, 0)

    run_pass_a(RAYS_PER_W - 1, (RING - 1) % RING)
    fire(RING - 1)
    for k in range(RING):
        wait_pair(k)
        run_pass_b(RAYS_PER_W - RING + k, k)

    pltpu.sync_copy(accb.at[pl.ds(0, OUTW)],
                    out_hbm.at[pl.ds(base_ray * N_POINTS, OUTW)])
    pltpu.sync_copy(hitsb, hits_hbm.at[pl.ds(base_ray, RAYS_PER_W)])
    pltpu.sync_copy(idxfb, idxf_hbm.at[pl.ds(base_ray, RAYS_PER_W)])


def _make_fwd():
    mesh = plsc.VectorSubcoreMesh(core_axis_name="c", subcore_axis_name="s")
    return pl.kernel(
        _sc_body,
        out_type=(
            jax.ShapeDtypeStruct((N_RAYS * N_POINTS,), jnp.float32),
            jax.ShapeDtypeStruct((N_RAYS,), jnp.float32),
            jax.ShapeDtypeStruct((N_RAYS,), jnp.int32),
        ),
        mesh=mesh,
        compiler_params=pltpu.CompilerParams(needs_layout_passes=False),
        scratch_types=[
            pltpu.VMEM((TBL012,), jnp.float32),
            pltpu.VMEM((RAYS_PER_W * 4 + 16,), jnp.float32),
            pltpu.VMEM((NPAD,), jnp.float32),
            pltpu.VMEM((16,), jnp.float32),
            pltpu.VMEM((8 * NPAD,), jnp.int32),
            pltpu.VMEM((2 * NPAD,), jnp.int32),
            pltpu.VMEM((5 * NPAD,), jnp.float32),
            pltpu.VMEM((RING * RL,), jnp.int32),
            pltpu.VMEM((RING * RL,), jnp.int32),
            pltpu.VMEM((RING * RL,), jnp.float32),
            pltpu.VMEM((RING * RL,), jnp.float32),
            pltpu.VMEM((RING * RL,), jnp.float32),
            pltpu.VMEM((RING * RL,), jnp.float32),
            pltpu.VMEM((OUTW + 16,), jnp.float32),
            pltpu.VMEM((RAYS_PER_W,), jnp.float32),
            pltpu.VMEM((RAYS_PER_W,), jnp.int32),
            pltpu.VMEM_SHARED((16 * 17168,), jnp.float32),
            pltpu.SemaphoreType.DMA((RING,)),
            pltpu.SemaphoreType.DMA((RING,)),
        ],
    )


def kernel(x, embeddings, W, b):
    emb = embeddings.reshape(-1)
    t_all = jnp.linspace(0.0, 1.0, N_POINTS, dtype=jnp.float32)
    t_pad = jnp.concatenate([t_all, jnp.broadcast_to(t_all[-1], (NPAD - N_POINTS,))])
    wb = jnp.concatenate([W[0], b, jnp.zeros((10,), jnp.float32)])
    out1d, hits, idxf = _make_fwd()(x.reshape(-1), emb, t_pad, wb)
    return (hits.reshape(N_RAYS, 1), out1d.reshape(N_RAYS, N_POINTS, 1), idxf)
